# baseline (speedup 1.0000x reference)
import numpy as np
import jax
import jax.numpy as jnp
from jax.experimental import pallas as pl

_NUM_MASKS = 2
_MAX_WIDTH = 40


def _span_mask(B, T):
    rng = np.random.RandomState(0)
    mask = np.ones((B, 1, T), dtype=np.float32)
    for b in range(B):
        for _ in range(_NUM_MASKS):
            width = int(rng.randint(1, _MAX_WIDTH + 1))
            if T - width <= 0:
                continue
            start = int(rng.randint(0, T - width))
            mask[b, 0, start:start + width] = 0.0
    return mask


def _mask_mul(x_ref, m_ref, o_ref):
    o_ref[...] = x_ref[...] * m_ref[...]


def kernel(x):
    B, C, T = x.shape
    mask = jnp.asarray(_span_mask(B, T))

    Tt = 640
    grid = (B, T // Tt)
    return pl.pallas_call(
        _mask_mul,
        grid=grid,
        in_specs=[
            pl.BlockSpec((1, C, Tt), lambda b, j: (b, 0, j)),
            pl.BlockSpec((1, 1, Tt), lambda b, j: (b, 0, j)),
        ],
        out_specs=pl.BlockSpec((1, C, Tt), lambda b, j: (b, 0, j)),
        out_shape=jax.ShapeDtypeStruct((B, C, T), x.dtype),
    )(x, mask)


# speedup vs baseline: 1.3718x; 1.3718x over previous
import numpy as np
import jax
import jax.numpy as jnp
from jax.experimental import pallas as pl

_NUM_MASKS = 2
_MAX_WIDTH = 40


def _span_mask(B, T):
    rng = np.random.RandomState(0)
    mask = np.ones((B, 1, T), dtype=np.float32)
    for b in range(B):
        for _ in range(_NUM_MASKS):
            width = int(rng.randint(1, _MAX_WIDTH + 1))
            if T - width <= 0:
                continue
            start = int(rng.randint(0, T - width))
            mask[b, 0, start:start + width] = 0.0
    return mask


def _mask_mul(x_ref, m_ref, o_ref):
    o_ref[...] = x_ref[...] * m_ref[...]


def kernel(x):
    B, C, T = x.shape
    mask = jnp.asarray(_span_mask(B, T))

    Ct = 128
    grid = (B, C // Ct)
    return pl.pallas_call(
        _mask_mul,
        grid=grid,
        in_specs=[
            pl.BlockSpec((1, Ct, T), lambda b, c: (b, c, 0)),
            pl.BlockSpec((1, 1, T), lambda b, c: (b, 0, 0)),
        ],
        out_specs=pl.BlockSpec((1, Ct, T), lambda b, c: (b, c, 0)),
        out_shape=jax.ShapeDtypeStruct((B, C, T), x.dtype),
    )(x, mask)
